# Initial kernel scaffold; baseline (speedup 1.0000x reference)
import jax
import jax.numpy as jnp
from jax.experimental import pallas as pl


def kernel(x, edge_index, emb, W_mp, b_mp, W_cls, b_cls):
    raise NotImplementedError("write your pallas kernel here")


# speedup vs baseline: 5.2816x; 5.2816x over previous
import functools

import jax
import jax.numpy as jnp
from jax import lax
from jax.experimental import pallas as pl
from jax.experimental.pallas import tpu as pltpu
from jax.experimental.pallas import tpu_sc as plsc

NC = 2
NS = 16
EMB_D = 128

N_NODES = 10000
NPAD = 10240
NPT = NPAD // NS
NCHUNK = 128
NFULL = NPT // NCHUNK

N_EDGES = 320000
EPT = N_EDGES // (NC * NS)
ECHUNK = 80
EFULL = EPT // ECHUNK


def _sc_body(xp, src, dst, emb, zer, comb_out, h_out,
             nidx, sidx, didx, rows, erows, zbuf, comb_sh, sem):
    c = lax.axis_index("c")
    s = lax.axis_index("s")

    pltpu.sync_copy(zer, zbuf)

    def phase_a(j, carry):
        nb = s * NPT + j * NCHUNK
        pltpu.sync_copy(xp.at[pl.ds(nb, NCHUNK)], nidx)
        pltpu.async_copy(emb.at[nidx], rows, sem).wait()
        pltpu.sync_copy(rows, h_out.at[pl.ds(nb, NCHUNK)])

        @pl.when(c == 0)
        def _():
            pltpu.sync_copy(rows, comb_sh.at[pl.ds(nb, NCHUNK)])

        @pl.when(c == 1)
        def _():
            pltpu.sync_copy(zbuf, comb_sh.at[pl.ds(nb, NCHUNK)])

        return carry
    lax.fori_loop(0, NFULL, phase_a, 0)
    plsc.subcore_barrier()

    eb = (c * NS + s) * EPT
    def phase_b(i, carry):
        off = pl.multiple_of(eb + i * ECHUNK, 8)
        pltpu.sync_copy(src.at[pl.ds(off, ECHUNK)], sidx)
        pltpu.sync_copy(dst.at[pl.ds(off, ECHUNK)], didx)
        pltpu.async_copy(h_out.at[sidx], erows, sem).wait()
        pltpu.sync_copy(erows, comb_sh.at[didx], add=True)
        return carry
    lax.fori_loop(0, EFULL, phase_b, 0)
    plsc.subcore_barrier()

    def phase_c(j, carry):
        nb = s * NPT + j * NCHUNK
        pltpu.sync_copy(comb_sh.at[pl.ds(nb, NCHUNK)], rows)
        pltpu.sync_copy(rows, comb_out.at[c, pl.ds(nb, NCHUNK)])
        return carry
    lax.fori_loop(0, NFULL, phase_c, 0)


_sc_gnn = functools.partial(
    pl.kernel,
    out_type=(
        jax.ShapeDtypeStruct((NC, NPAD, EMB_D), jnp.float32),
        jax.ShapeDtypeStruct((NPAD, EMB_D), jnp.float32),
    ),
    mesh=plsc.VectorSubcoreMesh(
        core_axis_name="c", subcore_axis_name="s",
        num_cores=NC, num_subcores=NS),
    scratch_types=[
        pltpu.VMEM((NCHUNK,), jnp.int32),
        pltpu.VMEM((ECHUNK,), jnp.int32),
        pltpu.VMEM((ECHUNK,), jnp.int32),
        pltpu.VMEM((NCHUNK, EMB_D), jnp.float32),
        pltpu.VMEM((ECHUNK, EMB_D), jnp.float32),
        pltpu.VMEM((NCHUNK, EMB_D), jnp.float32),
        pltpu.VMEM_SHARED((NPAD, EMB_D), jnp.float32),
        pltpu.SemaphoreType.DMA,
    ],
)(_sc_body)


BN = 1024
NBLK = NPAD // BN


def _tc_body(comb_ref, wmp_ref, bmp_ref, wcls_ref, bcls_ref, out_ref, acc_ref):
    i = pl.program_id(0)

    @pl.when(i == 0)
    def _():
        acc_ref[...] = jnp.zeros_like(acc_ref)

    cb = comb_ref[...]
    zin = cb[0] + cb[1]
    z = jax.lax.dot(zin, wmp_ref[...],
                    precision=jax.lax.Precision.HIGHEST,
                    preferred_element_type=jnp.float32)
    z = jnp.maximum(z + bmp_ref[...], 0.0)
    rid = i * BN + lax.broadcasted_iota(jnp.int32, (BN, 1), 0)
    z = jnp.where(rid < N_NODES, z, 0.0)
    acc_ref[...] += jnp.sum(z, axis=0, keepdims=True)

    @pl.when(i == NBLK - 1)
    def _():
        hg = acc_ref[...] * (1.0 / N_NODES)
        out_ref[...] = jax.lax.dot(
            hg, wcls_ref[...],
            precision=jax.lax.Precision.HIGHEST,
            preferred_element_type=jnp.float32) + bcls_ref[...]


def _tc_tail(comb, W_mp, b_mp, W_cls, b_cls):
    return pl.pallas_call(
        _tc_body,
        grid=(NBLK,),
        in_specs=[
            pl.BlockSpec((NC, BN, EMB_D), lambda i: (0, i, 0)),
            pl.BlockSpec((128, 128), lambda i: (0, 0)),
            pl.BlockSpec((1, 128), lambda i: (0, 0)),
            pl.BlockSpec((128, 16), lambda i: (0, 0)),
            pl.BlockSpec((1, 16), lambda i: (0, 0)),
        ],
        out_specs=pl.BlockSpec((1, 16), lambda i: (0, 0)),
        out_shape=jax.ShapeDtypeStruct((1, 16), jnp.float32),
        scratch_shapes=[pltpu.VMEM((1, 128), jnp.float32)],
    )(comb, W_mp, b_mp, W_cls, b_cls)


def kernel(x, edge_index, emb, W_mp, b_mp, W_cls, b_cls):
    x = x.astype(jnp.int32)
    pad = jnp.arange(NPAD - N_NODES, dtype=jnp.int32)
    xp = jnp.concatenate([x, pad])
    src = edge_index[0]
    dst = edge_index[1]
    zer = jnp.zeros((NCHUNK, EMB_D), dtype=jnp.float32)
    comb, _h = _sc_gnn(xp, src, dst, emb, zer)
    return _tc_tail(comb, W_mp, b_mp.reshape(1, 128), W_cls,
                    b_cls.reshape(1, 16))


# speedup vs baseline: 10.7346x; 2.0324x over previous
import functools

import jax
import jax.numpy as jnp
from jax import lax
from jax.experimental import pallas as pl
from jax.experimental.pallas import tpu as pltpu
from jax.experimental.pallas import tpu_sc as plsc

NC = 2
NS = 16
NW = NC * NS
EMB_D = 128

N_NODES = 10000
NPAD = 10240
NPT = NPAD // NS
NCHUNK = 128
NFULL = NPT // NCHUNK

N_EDGES = 320000
ECHUNK = 128
CPW = 79
EPAD = NW * CPW * ECHUNK
CTRIPLE = (CPW - 1) // 3


def _sc_body(xp, es, emb, zer, comb_out, h_out,
             nidx, erows_a, erows_b, eidx_a, eidx_b, comb_sh,
             asem, gsem_a, gsem_b, ssem_a, ssem_b):
    c = lax.axis_index("c")
    s = lax.axis_index("s")
    w = c * NS + s

    @pl.when(c == 1)
    def _():
        pltpu.sync_copy(zer, erows_a)

        def zloop(j, carry):
            nb = s * NPT + j * NCHUNK
            pltpu.sync_copy(erows_a, comb_sh.at[pl.ds(nb, NCHUNK)])
            return carry
        lax.fori_loop(0, NFULL, zloop, 0)

    def phase_a(j, carry):
        nb = s * NPT + j * NCHUNK
        pltpu.sync_copy(xp.at[pl.ds(nb, NCHUNK)], nidx)
        pltpu.async_copy(emb.at[nidx], erows_a, asem).wait()
        pltpu.sync_copy(erows_a, h_out.at[pl.ds(nb, NCHUNK)])

        @pl.when(c == 0)
        def _():
            pltpu.sync_copy(erows_a, comb_sh.at[pl.ds(nb, NCHUNK)])

        return carry
    lax.fori_loop(0, NFULL, phase_a, 0)
    plsc.subcore_barrier()

    cb = w * CPW

    def load_idx(eidx, i):
        pltpu.sync_copy(es.at[i], eidx)

    def start_gather(eidx, erows, gsem):
        pltpu.async_copy(h_out.at[eidx.at[0]], erows, gsem)

    def wait_gather(eidx, erows, gsem):
        pltpu.make_async_copy(h_out.at[eidx.at[0]], erows, gsem).wait()

    def start_scatter(eidx, erows, ssem):
        pltpu.async_copy(erows, comb_sh.at[eidx.at[1]], ssem, add=True)

    def wait_scatter(eidx, erows, ssem):
        pltpu.make_async_copy(erows, comb_sh.at[eidx.at[1]], ssem).wait()

    def pair_body(k, carry):
        @pl.when(k > 0)
        def _():
            wait_scatter(eidx_a, erows_a, ssem_a)
        load_idx(eidx_a, cb + 2 * k)
        start_gather(eidx_a, erows_a, gsem_a)

        @pl.when(k > 0)
        def _():
            wait_gather(eidx_b, erows_b, gsem_b)
            start_scatter(eidx_b, erows_b, ssem_b)

        @pl.when(k > 0)
        def _():
            wait_scatter(eidx_b, erows_b, ssem_b)
        load_idx(eidx_b, cb + 2 * k + 1)
        start_gather(eidx_b, erows_b, gsem_b)
        wait_gather(eidx_a, erows_a, gsem_a)
        start_scatter(eidx_a, erows_a, ssem_a)
        return carry

    lax.fori_loop(0, (CPW - 1) // 2, pair_body, 0)
    wait_scatter(eidx_a, erows_a, ssem_a)
    load_idx(eidx_a, cb + CPW - 1)
    start_gather(eidx_a, erows_a, gsem_a)
    wait_gather(eidx_b, erows_b, gsem_b)
    start_scatter(eidx_b, erows_b, ssem_b)
    wait_gather(eidx_a, erows_a, gsem_a)
    start_scatter(eidx_a, erows_a, ssem_a)
    wait_scatter(eidx_b, erows_b, ssem_b)
    wait_scatter(eidx_a, erows_a, ssem_a)
    plsc.subcore_barrier()

    def phase_c(j, carry):
        nb = s * NPT + j * NCHUNK
        pltpu.sync_copy(comb_sh.at[pl.ds(nb, NCHUNK)], erows_a)
        pltpu.sync_copy(erows_a, comb_out.at[c, pl.ds(nb, NCHUNK)])
        return carry
    lax.fori_loop(0, NFULL, phase_c, 0)


_sc_gnn = functools.partial(
    pl.kernel,
    out_type=(
        jax.ShapeDtypeStruct((NC, NPAD, EMB_D), jnp.float32),
        jax.ShapeDtypeStruct((NPAD, EMB_D), jnp.float32),
    ),
    mesh=plsc.VectorSubcoreMesh(
        core_axis_name="c", subcore_axis_name="s",
        num_cores=NC, num_subcores=NS),
    scratch_types=[
        pltpu.VMEM((NCHUNK,), jnp.int32),
        pltpu.VMEM((ECHUNK, EMB_D), jnp.float32),
        pltpu.VMEM((ECHUNK, EMB_D), jnp.float32),
        pltpu.VMEM((2, ECHUNK), jnp.int32),
        pltpu.VMEM((2, ECHUNK), jnp.int32),
        pltpu.VMEM_SHARED((NPAD, EMB_D), jnp.float32),
        pltpu.SemaphoreType.DMA,
        pltpu.SemaphoreType.DMA,
        pltpu.SemaphoreType.DMA,
        pltpu.SemaphoreType.DMA,
        pltpu.SemaphoreType.DMA,
    ],
)(_sc_body)


BN = 1024
NBLK = NPAD // BN


def _tc_body(comb_ref, wmp_ref, bmp_ref, wcls_ref, bcls_ref, out_ref, acc_ref):
    i = pl.program_id(0)

    @pl.when(i == 0)
    def _():
        acc_ref[...] = jnp.zeros_like(acc_ref)

    cb = comb_ref[...]
    zin = cb[0] + cb[1]
    z = jax.lax.dot(zin, wmp_ref[...],
                    precision=jax.lax.Precision.HIGHEST,
                    preferred_element_type=jnp.float32)
    z = jnp.maximum(z + bmp_ref[...], 0.0)
    rid = i * BN + lax.broadcasted_iota(jnp.int32, (BN, 1), 0)
    z = jnp.where(rid < N_NODES, z, 0.0)
    acc_ref[...] += jnp.sum(z, axis=0, keepdims=True)

    @pl.when(i == NBLK - 1)
    def _():
        hg = acc_ref[...] * (1.0 / N_NODES)
        out_ref[...] = jax.lax.dot(
            hg, wcls_ref[...],
            precision=jax.lax.Precision.HIGHEST,
            preferred_element_type=jnp.float32) + bcls_ref[...]


def _tc_tail(comb, W_mp, b_mp, W_cls, b_cls):
    return pl.pallas_call(
        _tc_body,
        grid=(NBLK,),
        in_specs=[
            pl.BlockSpec((NC, BN, EMB_D), lambda i: (0, i, 0)),
            pl.BlockSpec((128, 128), lambda i: (0, 0)),
            pl.BlockSpec((1, 128), lambda i: (0, 0)),
            pl.BlockSpec((128, 16), lambda i: (0, 0)),
            pl.BlockSpec((1, 16), lambda i: (0, 0)),
        ],
        out_specs=pl.BlockSpec((1, 16), lambda i: (0, 0)),
        out_shape=jax.ShapeDtypeStruct((1, 16), jnp.float32),
        scratch_shapes=[pltpu.VMEM((1, 128), jnp.float32)],
    )(comb, W_mp, b_mp, W_cls, b_cls)


def kernel(x, edge_index, emb, W_mp, b_mp, W_cls, b_cls):
    x = x.astype(jnp.int32)
    pad = jnp.arange(NPAD - N_NODES, dtype=jnp.int32)
    xp = jnp.concatenate([x, pad])
    npe = EPAD - N_EDGES
    pe = jnp.arange(npe, dtype=jnp.int32)
    src = jnp.concatenate([edge_index[0], pe % N_NODES])
    dst = jnp.concatenate([edge_index[1], N_NODES + pe % (NPAD - N_NODES)])
    es = jnp.stack([src, dst]).reshape(2, EPAD // ECHUNK, ECHUNK)
    es = es.transpose(1, 0, 2)
    zer = jnp.zeros((NCHUNK, EMB_D), dtype=jnp.float32)
    comb, _h = _sc_gnn(xp, es, emb, zer)
    return _tc_tail(comb, W_mp, b_mp.reshape(1, 128), W_cls,
                    b_cls.reshape(1, 16))


# speedup vs baseline: 11.6705x; 1.0872x over previous
import functools

import jax
import jax.numpy as jnp
from jax import lax
from jax.experimental import pallas as pl
from jax.experimental.pallas import tpu as pltpu
from jax.experimental.pallas import tpu_sc as plsc

NC = 2
NS = 16
NW = NC * NS
EMB_D = 128

N_NODES = 10000
NPAD = 10240
NPT = NPAD // NS
NCHUNK = 128
NFULL = NPT // NCHUNK

N_EDGES = 320000
ECHUNK = 128
GPW = 20
CPW = GPW * 4
EPAD = NW * CPW * ECHUNK
NGRP = NW * GPW


def _sc_body(xp, es, emb, zer, comb_out, h_out,
             nidx, erows_a, erows_b, eidx_a, eidx_b, comb_sh,
             asem, isem_a, isem_b, gsem_a, gsem_b, ssem_a, ssem_b):
    c = lax.axis_index("c")
    s = lax.axis_index("s")
    w = c * NS + s

    @pl.when(c == 1)
    def _():
        pltpu.sync_copy(zer, erows_a)

        def zloop(j, carry):
            nb = s * NPT + j * NCHUNK
            pltpu.sync_copy(erows_a, comb_sh.at[pl.ds(nb, NCHUNK)])
            return carry
        lax.fori_loop(0, NFULL, zloop, 0)

    def phase_a(j, carry):
        nb = s * NPT + j * NCHUNK
        pltpu.sync_copy(xp.at[pl.ds(nb, NCHUNK)], nidx)
        pltpu.async_copy(emb.at[nidx], erows_a, asem).wait()
        pltpu.sync_copy(erows_a, h_out.at[pl.ds(nb, NCHUNK)])

        @pl.when(c == 0)
        def _():
            pltpu.sync_copy(erows_a, comb_sh.at[pl.ds(nb, NCHUNK)])

        return carry
    lax.fori_loop(0, NFULL, phase_a, 0)
    plsc.subcore_barrier()

    gb = w * GPW

    def load_grp(eidx, g, isem):
        pltpu.async_copy(es.at[g], eidx, isem)

    def wait_grp(eidx, g, isem):
        pltpu.make_async_copy(es.at[g], eidx, isem).wait()

    def start_gather(eidx, r, erows, gsem):
        pltpu.async_copy(h_out.at[eidx.at[r]], erows, gsem)

    def wait_gather(erows, gsem):
        pltpu.make_async_copy(h_out.at[eidx_a.at[0]], erows, gsem).wait()

    def start_scatter(erows, eidx, r, ssem):
        pltpu.async_copy(erows, comb_sh.at[eidx.at[r]], ssem, add=True)

    def wait_scatter(erows, ssem):
        pltpu.make_async_copy(erows, comb_sh.at[eidx_a.at[1]], ssem).wait()

    load_grp(eidx_a, gb, isem_a)
    NBODY = GPW // 2

    def grp_body(k, carry):
        g0 = gb + 2 * k
        @pl.when(k > 0)
        def _():
            wait_scatter(erows_a, ssem_a)
        wait_grp(eidx_a, g0, isem_a)
        start_gather(eidx_a, 0, erows_a, gsem_a)

        @pl.when(k > 0)
        def _():
            wait_gather(erows_b, gsem_b)
            start_scatter(erows_b, eidx_b, 7, ssem_b)

        @pl.when(k > 0)
        def _():
            wait_scatter(erows_b, ssem_b)
        load_grp(eidx_b, g0 + 1, isem_b)
        start_gather(eidx_a, 2, erows_b, gsem_b)
        wait_gather(erows_a, gsem_a)
        start_scatter(erows_a, eidx_a, 1, ssem_a)

        wait_scatter(erows_a, ssem_a)
        start_gather(eidx_a, 4, erows_a, gsem_a)
        wait_gather(erows_b, gsem_b)
        start_scatter(erows_b, eidx_a, 3, ssem_b)

        wait_scatter(erows_b, ssem_b)
        start_gather(eidx_a, 6, erows_b, gsem_b)
        wait_gather(erows_a, gsem_a)
        start_scatter(erows_a, eidx_a, 5, ssem_a)

        wait_scatter(erows_a, ssem_a)
        wait_grp(eidx_b, g0 + 1, isem_b)
        start_gather(eidx_b, 0, erows_a, gsem_a)
        wait_gather(erows_b, gsem_b)
        start_scatter(erows_b, eidx_a, 7, ssem_b)

        wait_scatter(erows_b, ssem_b)

        @pl.when(k < NBODY - 1)
        def _():
            load_grp(eidx_a, g0 + 2, isem_a)
        start_gather(eidx_b, 2, erows_b, gsem_b)
        wait_gather(erows_a, gsem_a)
        start_scatter(erows_a, eidx_b, 1, ssem_a)

        wait_scatter(erows_a, ssem_a)
        start_gather(eidx_b, 4, erows_a, gsem_a)
        wait_gather(erows_b, gsem_b)
        start_scatter(erows_b, eidx_b, 3, ssem_b)

        wait_scatter(erows_b, ssem_b)
        start_gather(eidx_b, 6, erows_b, gsem_b)
        wait_gather(erows_a, gsem_a)
        start_scatter(erows_a, eidx_b, 5, ssem_a)
        return carry

    lax.fori_loop(0, NBODY, grp_body, 0)
    wait_gather(erows_b, gsem_b)
    start_scatter(erows_b, eidx_b, 7, ssem_b)
    wait_scatter(erows_a, ssem_a)
    wait_scatter(erows_b, ssem_b)
    plsc.subcore_barrier()

    def phase_c(j, carry):
        nb = s * NPT + j * NCHUNK
        pltpu.sync_copy(comb_sh.at[pl.ds(nb, NCHUNK)], erows_a)
        pltpu.sync_copy(erows_a, comb_out.at[c, pl.ds(nb, NCHUNK)])
        return carry
    lax.fori_loop(0, NFULL, phase_c, 0)


_sc_gnn = functools.partial(
    pl.kernel,
    out_type=(
        jax.ShapeDtypeStruct((NC, NPAD, EMB_D), jnp.float32),
        jax.ShapeDtypeStruct((NPAD, EMB_D), jnp.float32),
    ),
    mesh=plsc.VectorSubcoreMesh(
        core_axis_name="c", subcore_axis_name="s",
        num_cores=NC, num_subcores=NS),
    scratch_types=[
        pltpu.VMEM((NCHUNK,), jnp.int32),
        pltpu.VMEM((ECHUNK, EMB_D), jnp.float32),
        pltpu.VMEM((ECHUNK, EMB_D), jnp.float32),
        pltpu.VMEM((8, ECHUNK), jnp.int32),
        pltpu.VMEM((8, ECHUNK), jnp.int32),
        pltpu.VMEM_SHARED((NPAD, EMB_D), jnp.float32),
        pltpu.SemaphoreType.DMA,
        pltpu.SemaphoreType.DMA,
        pltpu.SemaphoreType.DMA,
        pltpu.SemaphoreType.DMA,
        pltpu.SemaphoreType.DMA,
        pltpu.SemaphoreType.DMA,
        pltpu.SemaphoreType.DMA,
    ],
)(_sc_body)


BN = 1024
NBLK = NPAD // BN


def _tc_body(comb_ref, wmp_ref, bmp_ref, wcls_ref, bcls_ref, out_ref, acc_ref):
    i = pl.program_id(0)

    @pl.when(i == 0)
    def _():
        acc_ref[...] = jnp.zeros_like(acc_ref)

    cb = comb_ref[...]
    zin = cb[0] + cb[1]
    z = jax.lax.dot(zin, wmp_ref[...],
                    precision=jax.lax.Precision.HIGHEST,
                    preferred_element_type=jnp.float32)
    z = jnp.maximum(z + bmp_ref[...], 0.0)
    rid = i * BN + lax.broadcasted_iota(jnp.int32, (BN, 1), 0)
    z = jnp.where(rid < N_NODES, z, 0.0)
    acc_ref[...] += jnp.sum(z, axis=0, keepdims=True)

    @pl.when(i == NBLK - 1)
    def _():
        hg = acc_ref[...] * (1.0 / N_NODES)
        out_ref[...] = jax.lax.dot(
            hg, wcls_ref[...],
            precision=jax.lax.Precision.HIGHEST,
            preferred_element_type=jnp.float32) + bcls_ref[...]


def _tc_tail(comb, W_mp, b_mp, W_cls, b_cls):
    return pl.pallas_call(
        _tc_body,
        grid=(NBLK,),
        in_specs=[
            pl.BlockSpec((NC, BN, EMB_D), lambda i: (0, i, 0)),
            pl.BlockSpec((128, 128), lambda i: (0, 0)),
            pl.BlockSpec((1, 128), lambda i: (0, 0)),
            pl.BlockSpec((128, 16), lambda i: (0, 0)),
            pl.BlockSpec((1, 16), lambda i: (0, 0)),
        ],
        out_specs=pl.BlockSpec((1, 16), lambda i: (0, 0)),
        out_shape=jax.ShapeDtypeStruct((1, 16), jnp.float32),
        scratch_shapes=[pltpu.VMEM((1, 128), jnp.float32)],
    )(comb, W_mp, b_mp, W_cls, b_cls)


def kernel(x, edge_index, emb, W_mp, b_mp, W_cls, b_cls):
    x = x.astype(jnp.int32)
    pad = jnp.arange(NPAD - N_NODES, dtype=jnp.int32)
    xp = jnp.concatenate([x, pad])
    npe = EPAD - N_EDGES
    pe = jnp.arange(npe, dtype=jnp.int32)
    src = jnp.concatenate([edge_index[0], pe % N_NODES])
    dst = jnp.concatenate([edge_index[1], N_NODES + pe % (NPAD - N_NODES)])
    es = jnp.stack([src.reshape(-1, ECHUNK), dst.reshape(-1, ECHUNK)],
                   axis=1)
    es = es.reshape(NGRP, 8, ECHUNK)
    zer = jnp.zeros((NCHUNK, EMB_D), dtype=jnp.float32)
    comb, _h = _sc_gnn(xp, es, emb, zer)
    return _tc_tail(comb, W_mp, b_mp.reshape(1, 128), W_cls,
                    b_cls.reshape(1, 16))


# speedup vs baseline: 12.3266x; 1.0562x over previous
import functools

import jax
import jax.numpy as jnp
from jax import lax
from jax.experimental import pallas as pl
from jax.experimental.pallas import tpu as pltpu
from jax.experimental.pallas import tpu_sc as plsc

NC = 2
NS = 16
NW = NC * NS
EMB_D = 128

N_NODES = 10000
NPAD = 10240
NPT = NPAD // NS
NCHUNK = 128
NFULL = NPT // NCHUNK

N_EDGES = 320000
ECHUNK = 128
GPW = 20
CPW = GPW * 4
EPAD = NW * CPW * ECHUNK
NGRP = NW * GPW


def _sc_body(xp, es, emb, zer, comb_out, h_out,
             nidx, erows_a, erows_b, sidx_a, didx_a, sidx_b, didx_b,
             comb_sh,
             asem, isem_a, isem_b, gsem_a, gsem_b, ssem_a, ssem_b):
    c = lax.axis_index("c")
    s = lax.axis_index("s")
    w = c * NS + s

    @pl.when(c == 1)
    def _():
        pltpu.sync_copy(zer, erows_a)

        def zloop(j, carry):
            nb = s * NPT + j * NCHUNK
            pltpu.sync_copy(erows_a, comb_sh.at[pl.ds(nb, NCHUNK)])
            return carry
        lax.fori_loop(0, NFULL, zloop, 0)

    def phase_a(j, carry):
        nb = s * NPT + j * NCHUNK
        pltpu.sync_copy(xp.at[pl.ds(nb, NCHUNK)], nidx)
        pltpu.async_copy(emb.at[nidx], erows_a, asem).wait()
        pltpu.sync_copy(erows_a, h_out.at[pl.ds(nb, NCHUNK)])

        @pl.when(c == 0)
        def _():
            pltpu.sync_copy(erows_a, comb_sh.at[pl.ds(nb, NCHUNK)])

        return carry
    lax.fori_loop(0, NFULL, phase_a, 0)
    plsc.subcore_barrier()

    gb = w * GPW

    def load_grp(sidx, didx, g, isem):
        pltpu.async_copy(es.at[0, pl.ds(g * 4, 4)], sidx, isem)
        pltpu.async_copy(es.at[1, pl.ds(g * 4, 4)], didx, isem)

    def wait_grp(sidx, didx, g, isem):
        pltpu.make_async_copy(es.at[0, pl.ds(g * 4, 4)], sidx, isem).wait()
        pltpu.make_async_copy(es.at[1, pl.ds(g * 4, 4)], didx, isem).wait()

    def start_gather(sidx, r, erows, gsem):
        pltpu.async_copy(h_out.at[sidx.at[r]], erows, gsem)

    def wait_gather(erows, gsem):
        pltpu.make_async_copy(h_out.at[sidx_a.at[0]], erows, gsem).wait()

    def start_scatter(erows, didx, r, ssem):
        pltpu.async_copy(erows, comb_sh.at[didx.at[r]], ssem, add=True)

    def wait_scatter(erows, ssem):
        pltpu.make_async_copy(erows, comb_sh.at[didx_a.at[0]], ssem).wait()

    load_grp(sidx_a, didx_a, gb, isem_a)
    NBODY = GPW // 2

    def grp_body(k, carry):
        g0 = gb + 2 * k
        @pl.when(k > 0)
        def _():
            wait_scatter(erows_a, ssem_a)
        wait_grp(sidx_a, didx_a, g0, isem_a)
        start_gather(sidx_a, 0, erows_a, gsem_a)

        @pl.when(k > 0)
        def _():
            wait_gather(erows_b, gsem_b)
            start_scatter(erows_b, didx_b, 3, ssem_b)

        @pl.when(k > 0)
        def _():
            wait_scatter(erows_b, ssem_b)
        load_grp(sidx_b, didx_b, g0 + 1, isem_b)
        start_gather(sidx_a, 1, erows_b, gsem_b)
        wait_gather(erows_a, gsem_a)
        start_scatter(erows_a, didx_a, 0, ssem_a)

        wait_scatter(erows_a, ssem_a)
        start_gather(sidx_a, 2, erows_a, gsem_a)
        wait_gather(erows_b, gsem_b)
        start_scatter(erows_b, didx_a, 1, ssem_b)

        wait_scatter(erows_b, ssem_b)
        start_gather(sidx_a, 3, erows_b, gsem_b)
        wait_gather(erows_a, gsem_a)
        start_scatter(erows_a, didx_a, 2, ssem_a)

        wait_scatter(erows_a, ssem_a)
        wait_grp(sidx_b, didx_b, g0 + 1, isem_b)
        start_gather(sidx_b, 0, erows_a, gsem_a)
        wait_gather(erows_b, gsem_b)
        start_scatter(erows_b, didx_a, 3, ssem_b)

        wait_scatter(erows_b, ssem_b)

        @pl.when(k < NBODY - 1)
        def _():
            load_grp(sidx_a, didx_a, g0 + 2, isem_a)
        start_gather(sidx_b, 1, erows_b, gsem_b)
        wait_gather(erows_a, gsem_a)
        start_scatter(erows_a, didx_b, 0, ssem_a)

        wait_scatter(erows_a, ssem_a)
        start_gather(sidx_b, 2, erows_a, gsem_a)
        wait_gather(erows_b, gsem_b)
        start_scatter(erows_b, didx_b, 1, ssem_b)

        wait_scatter(erows_b, ssem_b)
        start_gather(sidx_b, 3, erows_b, gsem_b)
        wait_gather(erows_a, gsem_a)
        start_scatter(erows_a, didx_b, 2, ssem_a)
        return carry

    lax.fori_loop(0, NBODY, grp_body, 0)
    wait_gather(erows_b, gsem_b)
    start_scatter(erows_b, didx_b, 3, ssem_b)
    wait_scatter(erows_a, ssem_a)
    wait_scatter(erows_b, ssem_b)
    plsc.subcore_barrier()

    def phase_c(j, carry):
        nb = s * NPT + j * NCHUNK
        pltpu.sync_copy(comb_sh.at[pl.ds(nb, NCHUNK)], erows_a)
        pltpu.sync_copy(erows_a, comb_out.at[c, pl.ds(nb, NCHUNK)])
        return carry
    lax.fori_loop(0, NFULL, phase_c, 0)


_sc_gnn = functools.partial(
    pl.kernel,
    out_type=(
        jax.ShapeDtypeStruct((NC, NPAD, EMB_D), jnp.float32),
        jax.ShapeDtypeStruct((NPAD, EMB_D), jnp.float32),
    ),
    mesh=plsc.VectorSubcoreMesh(
        core_axis_name="c", subcore_axis_name="s",
        num_cores=NC, num_subcores=NS),
    scratch_types=[
        pltpu.VMEM((NCHUNK,), jnp.int32),
        pltpu.VMEM((ECHUNK, EMB_D), jnp.float32),
        pltpu.VMEM((ECHUNK, EMB_D), jnp.float32),
        pltpu.VMEM((4, ECHUNK), jnp.int32),
        pltpu.VMEM((4, ECHUNK), jnp.int32),
        pltpu.VMEM((4, ECHUNK), jnp.int32),
        pltpu.VMEM((4, ECHUNK), jnp.int32),
        pltpu.VMEM_SHARED((NPAD, EMB_D), jnp.float32),
        pltpu.SemaphoreType.DMA,
        pltpu.SemaphoreType.DMA,
        pltpu.SemaphoreType.DMA,
        pltpu.SemaphoreType.DMA,
        pltpu.SemaphoreType.DMA,
        pltpu.SemaphoreType.DMA,
        pltpu.SemaphoreType.DMA,
    ],
)(_sc_body)


BN = 1024
NBLK = NPAD // BN


def _tc_body(comb_ref, wmp_ref, bmp_ref, wcls_ref, bcls_ref, out_ref, acc_ref):
    i = pl.program_id(0)

    @pl.when(i == 0)
    def _():
        acc_ref[...] = jnp.zeros_like(acc_ref)

    cb = comb_ref[...]
    zin = cb[0] + cb[1]
    z = jax.lax.dot(zin, wmp_ref[...],
                    precision=jax.lax.Precision.HIGHEST,
                    preferred_element_type=jnp.float32)
    z = jnp.maximum(z + bmp_ref[...], 0.0)
    rid = i * BN + lax.broadcasted_iota(jnp.int32, (BN, 1), 0)
    z = jnp.where(rid < N_NODES, z, 0.0)
    acc_ref[...] += jnp.sum(z, axis=0, keepdims=True)

    @pl.when(i == NBLK - 1)
    def _():
        hg = acc_ref[...] * (1.0 / N_NODES)
        out_ref[...] = jax.lax.dot(
            hg, wcls_ref[...],
            precision=jax.lax.Precision.HIGHEST,
            preferred_element_type=jnp.float32) + bcls_ref[...]


def _tc_tail(comb, W_mp, b_mp, W_cls, b_cls):
    return pl.pallas_call(
        _tc_body,
        grid=(NBLK,),
        in_specs=[
            pl.BlockSpec((NC, BN, EMB_D), lambda i: (0, i, 0)),
            pl.BlockSpec((128, 128), lambda i: (0, 0)),
            pl.BlockSpec((1, 128), lambda i: (0, 0)),
            pl.BlockSpec((128, 16), lambda i: (0, 0)),
            pl.BlockSpec((1, 16), lambda i: (0, 0)),
        ],
        out_specs=pl.BlockSpec((1, 16), lambda i: (0, 0)),
        out_shape=jax.ShapeDtypeStruct((1, 16), jnp.float32),
        scratch_shapes=[pltpu.VMEM((1, 128), jnp.float32)],
    )(comb, W_mp, b_mp, W_cls, b_cls)


def kernel(x, edge_index, emb, W_mp, b_mp, W_cls, b_cls):
    x = x.astype(jnp.int32)
    pad = jnp.arange(NPAD - N_NODES, dtype=jnp.int32)
    xp = jnp.concatenate([x, pad])
    npe = EPAD - N_EDGES
    pe = jnp.arange(npe, dtype=jnp.int32)
    pads = jnp.stack([pe % N_NODES, N_NODES + pe % (NPAD - N_NODES)])
    es = jnp.concatenate([edge_index, pads], axis=1)
    es = es.reshape(2, EPAD // ECHUNK, ECHUNK)
    zer = jnp.zeros((NCHUNK, EMB_D), dtype=jnp.float32)
    comb, _h = _sc_gnn(xp, es, emb, zer)
    return _tc_tail(comb, W_mp, b_mp.reshape(1, 128), W_cls,
                    b_cls.reshape(1, 16))


# speedup vs baseline: 12.5102x; 1.0149x over previous
import functools

import jax
import jax.numpy as jnp
from jax import lax
from jax.experimental import pallas as pl
from jax.experimental.pallas import tpu as pltpu
from jax.experimental.pallas import tpu_sc as plsc

NC = 2
NS = 16
NW = NC * NS
EMB_D = 128

N_NODES = 10000
NPAD = 10240
NPT = NPAD // NS
NCHUNK = 128
NFULL = NPT // NCHUNK

N_EDGES = 320000
ECHUNK = 128
GPW = 20
CPW = GPW * 4
EPAD = NW * CPW * ECHUNK
NGRP = NW * GPW


def _sc_body(xp, es, emb, zer, comb_out, h_out,
             nidx, erows_a, erows_b,
             sidx_a, didx_a, sidx_b, didx_b,
             comb_sh,
             asem, bsem, isem_a, isem_b, gsem_a, gsem_b, ssem_a, ssem_b):
    c = lax.axis_index("c")
    s = lax.axis_index("s")
    w = c * NS + s

    @pl.when(c == 1)
    def _():
        pltpu.sync_copy(zer, erows_a)

        def zloop(j, carry):
            nb = s * NPT + j * NCHUNK
            pltpu.sync_copy(erows_a, comb_sh.at[pl.ds(nb, NCHUNK)])
            return carry
        lax.fori_loop(0, NFULL, zloop, 0)

    abufs = (erows_b, asem), (erows_a, bsem)
    pltpu.sync_copy(xp.at[pl.ds(s * NPT, NCHUNK)], nidx)
    pltpu.async_copy(emb.at[nidx], erows_b, asem)
    for j in range(NFULL):
        rows, sem = abufs[j % 2]
        pltpu.make_async_copy(emb.at[nidx], rows, sem).wait()
        if j + 1 < NFULL:
            nrows, nsem = abufs[(j + 1) % 2]
            pltpu.sync_copy(xp.at[pl.ds(s * NPT + (j + 1) * NCHUNK, NCHUNK)],
                            nidx)
            pltpu.async_copy(emb.at[nidx], nrows, nsem)
        nb = s * NPT + j * NCHUNK
        pltpu.sync_copy(rows, h_out.at[pl.ds(nb, NCHUNK)])

        @pl.when(c == 0)
        def _():
            pltpu.sync_copy(rows, comb_sh.at[pl.ds(nb, NCHUNK)])

    plsc.subcore_barrier()

    gb = w * GPW

    def load_grp(sidx, didx, g, isem):
        pltpu.async_copy(es.at[0, pl.ds(g * 4, 4)], sidx, isem)
        pltpu.async_copy(es.at[1, pl.ds(g * 4, 4)], didx, isem)

    def wait_grp(sidx, didx, g, isem):
        pltpu.make_async_copy(es.at[0, pl.ds(g * 4, 4)], sidx, isem).wait()
        pltpu.make_async_copy(es.at[1, pl.ds(g * 4, 4)], didx, isem).wait()

    def start_gather(sidx, r, erows, gsem):
        pltpu.async_copy(h_out.at[sidx.at[r]], erows, gsem)

    def wait_gather(erows, gsem):
        pltpu.make_async_copy(h_out.at[sidx_a.at[0]], erows, gsem).wait()

    def start_scatter(erows, didx, r, ssem):
        pltpu.async_copy(erows, comb_sh.at[didx.at[r]], ssem, add=True)

    def wait_scatter(erows, ssem):
        pltpu.make_async_copy(erows, comb_sh.at[didx_a.at[0]], ssem).wait()

    load_grp(sidx_a, didx_a, gb, isem_a)
    NBODY = GPW // 2

    def grp_body(k, carry):
        g0 = gb + 2 * k
        @pl.when(k > 0)
        def _():
            wait_scatter(erows_a, ssem_a)
        wait_grp(sidx_a, didx_a, g0, isem_a)
        start_gather(sidx_a, 0, erows_a, gsem_a)

        @pl.when(k > 0)
        def _():
            wait_gather(erows_b, gsem_b)
            start_scatter(erows_b, didx_b, 3, ssem_b)

        @pl.when(k > 0)
        def _():
            wait_scatter(erows_b, ssem_b)
        load_grp(sidx_b, didx_b, g0 + 1, isem_b)
        start_gather(sidx_a, 1, erows_b, gsem_b)
        wait_gather(erows_a, gsem_a)
        start_scatter(erows_a, didx_a, 0, ssem_a)

        wait_scatter(erows_a, ssem_a)
        start_gather(sidx_a, 2, erows_a, gsem_a)
        wait_gather(erows_b, gsem_b)
        start_scatter(erows_b, didx_a, 1, ssem_b)

        wait_scatter(erows_b, ssem_b)
        start_gather(sidx_a, 3, erows_b, gsem_b)
        wait_gather(erows_a, gsem_a)
        start_scatter(erows_a, didx_a, 2, ssem_a)

        wait_scatter(erows_a, ssem_a)
        wait_grp(sidx_b, didx_b, g0 + 1, isem_b)
        start_gather(sidx_b, 0, erows_a, gsem_a)
        wait_gather(erows_b, gsem_b)
        start_scatter(erows_b, didx_a, 3, ssem_b)

        wait_scatter(erows_b, ssem_b)

        @pl.when(k < NBODY - 1)
        def _():
            load_grp(sidx_a, didx_a, g0 + 2, isem_a)
        start_gather(sidx_b, 1, erows_b, gsem_b)
        wait_gather(erows_a, gsem_a)
        start_scatter(erows_a, didx_b, 0, ssem_a)

        wait_scatter(erows_a, ssem_a)
        start_gather(sidx_b, 2, erows_a, gsem_a)
        wait_gather(erows_b, gsem_b)
        start_scatter(erows_b, didx_b, 1, ssem_b)

        wait_scatter(erows_b, ssem_b)
        start_gather(sidx_b, 3, erows_b, gsem_b)
        wait_gather(erows_a, gsem_a)
        start_scatter(erows_a, didx_b, 2, ssem_a)
        return carry

    lax.fori_loop(0, NBODY, grp_body, 0)
    wait_gather(erows_b, gsem_b)
    start_scatter(erows_b, didx_b, 3, ssem_b)
    wait_scatter(erows_a, ssem_a)
    wait_scatter(erows_b, ssem_b)
    plsc.subcore_barrier()

    def phase_c(j, carry):
        nb = s * NPT + j * NCHUNK
        pltpu.sync_copy(comb_sh.at[pl.ds(nb, NCHUNK)], erows_a)
        pltpu.sync_copy(erows_a, comb_out.at[c, pl.ds(nb, NCHUNK)])
        return carry
    lax.fori_loop(0, NFULL, phase_c, 0)


_sc_gnn = functools.partial(
    pl.kernel,
    out_type=(
        jax.ShapeDtypeStruct((NC, NPAD, EMB_D), jnp.float32),
        jax.ShapeDtypeStruct((NPAD, EMB_D), jnp.float32),
    ),
    mesh=plsc.VectorSubcoreMesh(
        core_axis_name="c", subcore_axis_name="s",
        num_cores=NC, num_subcores=NS),
    scratch_types=[
        pltpu.VMEM((NCHUNK,), jnp.int32),
        pltpu.VMEM((ECHUNK, EMB_D), jnp.float32),
        pltpu.VMEM((ECHUNK, EMB_D), jnp.float32),
        pltpu.VMEM((4, ECHUNK), jnp.int32),
        pltpu.VMEM((4, ECHUNK), jnp.int32),
        pltpu.VMEM((4, ECHUNK), jnp.int32),
        pltpu.VMEM((4, ECHUNK), jnp.int32),
        pltpu.VMEM_SHARED((NPAD, EMB_D), jnp.float32),
        pltpu.SemaphoreType.DMA,
        pltpu.SemaphoreType.DMA,
        pltpu.SemaphoreType.DMA,
        pltpu.SemaphoreType.DMA,
        pltpu.SemaphoreType.DMA,
        pltpu.SemaphoreType.DMA,
        pltpu.SemaphoreType.DMA,
        pltpu.SemaphoreType.DMA,
    ],
)(_sc_body)


BN = 1024
NBLK = NPAD // BN


def _tc_body(comb_ref, wmp_ref, bmp_ref, wcls_ref, bcls_ref, out_ref, acc_ref):
    i = pl.program_id(0)

    @pl.when(i == 0)
    def _():
        acc_ref[...] = jnp.zeros_like(acc_ref)

    cb = comb_ref[...]
    zin = cb[0] + cb[1]
    z = jax.lax.dot(zin, wmp_ref[...],
                    precision=jax.lax.Precision.HIGHEST,
                    preferred_element_type=jnp.float32)
    z = jnp.maximum(z + bmp_ref[...], 0.0)
    rid = i * BN + lax.broadcasted_iota(jnp.int32, (BN, 1), 0)
    z = jnp.where(rid < N_NODES, z, 0.0)
    acc_ref[...] += jnp.sum(z, axis=0, keepdims=True)

    @pl.when(i == NBLK - 1)
    def _():
        hg = acc_ref[...] * (1.0 / N_NODES)
        out_ref[...] = jax.lax.dot(
            hg, wcls_ref[...],
            precision=jax.lax.Precision.HIGHEST,
            preferred_element_type=jnp.float32) + bcls_ref[...]


def _tc_tail(comb, W_mp, b_mp, W_cls, b_cls):
    return pl.pallas_call(
        _tc_body,
        grid=(NBLK,),
        in_specs=[
            pl.BlockSpec((NC, BN, EMB_D), lambda i: (0, i, 0)),
            pl.BlockSpec((128, 128), lambda i: (0, 0)),
            pl.BlockSpec((1, 128), lambda i: (0, 0)),
            pl.BlockSpec((128, 16), lambda i: (0, 0)),
            pl.BlockSpec((1, 16), lambda i: (0, 0)),
        ],
        out_specs=pl.BlockSpec((1, 16), lambda i: (0, 0)),
        out_shape=jax.ShapeDtypeStruct((1, 16), jnp.float32),
        scratch_shapes=[pltpu.VMEM((1, 128), jnp.float32)],
    )(comb, W_mp, b_mp, W_cls, b_cls)


def kernel(x, edge_index, emb, W_mp, b_mp, W_cls, b_cls):
    x = x.astype(jnp.int32)
    pad = jnp.arange(NPAD - N_NODES, dtype=jnp.int32)
    xp = jnp.concatenate([x, pad])
    npe = EPAD - N_EDGES
    pe = jnp.arange(npe, dtype=jnp.int32)
    pads = jnp.stack([pe % N_NODES, N_NODES + pe % (NPAD - N_NODES)])
    es = jnp.concatenate([edge_index, pads], axis=1)
    es = es.reshape(2, EPAD // ECHUNK, ECHUNK)
    zer = jnp.zeros((NCHUNK, EMB_D), dtype=jnp.float32)
    comb, _h = _sc_gnn(xp, es, emb, zer)
    return _tc_tail(comb, W_mp, b_mp.reshape(1, 128), W_cls,
                    b_cls.reshape(1, 16))


# speedup vs baseline: 12.7658x; 1.0204x over previous
import functools

import jax
import jax.numpy as jnp
from jax import lax
from jax.experimental import pallas as pl
from jax.experimental.pallas import tpu as pltpu
from jax.experimental.pallas import tpu_sc as plsc

NC = 2
NS = 16
NW = NC * NS
EMB_D = 128

N_NODES = 10000
NPAD = 10240
NPT = NPAD // NS
NCHUNK = 128
NFULL = NPT // NCHUNK

N_EDGES = 320000
ECHUNK = 128
GPW = 20
CPW = GPW * 4
EPAD = NW * CPW * ECHUNK
NGRP = NW * GPW


def _sc_body(xp, es, emb, zer, comb_out, h_out,
             nidx, erows_a, erows_b,
             sidx_a, didx_a, sidx_b, didx_b,
             comb_sh,
             asem, bsem, isem_a, isem_b, gsem_a, gsem_b, ssem_a, ssem_b):
    c = lax.axis_index("c")
    s = lax.axis_index("s")
    w = c * NS + s

    @pl.when(c == 1)
    def _():
        pltpu.sync_copy(zer, erows_a)

        def zloop(j, carry):
            nb = s * NPT + j * NCHUNK
            pltpu.sync_copy(erows_a, comb_sh.at[pl.ds(nb, NCHUNK)])
            return carry
        lax.fori_loop(0, NFULL, zloop, 0)

    abufs = (erows_b, asem), (erows_a, bsem)
    pltpu.sync_copy(xp.at[pl.ds(s * NPT, NCHUNK)], nidx)
    pltpu.async_copy(emb.at[nidx], erows_b, asem)
    for j in range(NFULL):
        rows, sem = abufs[j % 2]
        pltpu.make_async_copy(emb.at[nidx], rows, sem).wait()
        if j + 1 < NFULL:
            nrows, nsem = abufs[(j + 1) % 2]
            pltpu.sync_copy(xp.at[pl.ds(s * NPT + (j + 1) * NCHUNK, NCHUNK)],
                            nidx)
            pltpu.async_copy(emb.at[nidx], nrows, nsem)
        nb = s * NPT + j * NCHUNK
        pltpu.sync_copy(rows, h_out.at[pl.ds(nb, NCHUNK)])

        @pl.when(c == 0)
        def _():
            pltpu.sync_copy(rows, comb_sh.at[pl.ds(nb, NCHUNK)])

    plsc.subcore_barrier()

    gb = w * GPW

    def load_grp(sidx, didx, g, isem):
        pltpu.async_copy(es.at[0, pl.ds(g * 4, 4)], sidx, isem)
        pltpu.async_copy(es.at[1, pl.ds(g * 4, 4)], didx, isem)

    def wait_grp(sidx, didx, g, isem):
        pltpu.make_async_copy(es.at[0, pl.ds(g * 4, 4)], sidx, isem).wait()
        pltpu.make_async_copy(es.at[1, pl.ds(g * 4, 4)], didx, isem).wait()

    def start_gather(sidx, r, erows, gsem):
        pltpu.async_copy(h_out.at[sidx.at[r]], erows, gsem)

    def wait_gather(erows, gsem):
        pltpu.make_async_copy(h_out.at[sidx_a.at[0]], erows, gsem).wait()

    def start_scatter(erows, didx, r, ssem):
        pltpu.async_copy(erows, comb_sh.at[didx.at[r]], ssem, add=True)

    def wait_scatter(erows, ssem):
        pltpu.make_async_copy(erows, comb_sh.at[didx_a.at[0]], ssem).wait()

    load_grp(sidx_a, didx_a, gb, isem_a)
    NBODY = GPW // 2

    def grp_body(k, carry):
        g0 = gb + 2 * k
        @pl.when(k > 0)
        def _():
            wait_scatter(erows_a, ssem_a)
        wait_grp(sidx_a, didx_a, g0, isem_a)
        start_gather(sidx_a, 0, erows_a, gsem_a)

        @pl.when(k > 0)
        def _():
            wait_gather(erows_b, gsem_b)
            start_scatter(erows_b, didx_b, 3, ssem_b)

        @pl.when(k > 0)
        def _():
            wait_scatter(erows_b, ssem_b)
        load_grp(sidx_b, didx_b, g0 + 1, isem_b)
        start_gather(sidx_a, 1, erows_b, gsem_b)
        wait_gather(erows_a, gsem_a)
        start_scatter(erows_a, didx_a, 0, ssem_a)

        wait_scatter(erows_a, ssem_a)
        start_gather(sidx_a, 2, erows_a, gsem_a)
        wait_gather(erows_b, gsem_b)
        start_scatter(erows_b, didx_a, 1, ssem_b)

        wait_scatter(erows_b, ssem_b)
        start_gather(sidx_a, 3, erows_b, gsem_b)
        wait_gather(erows_a, gsem_a)
        start_scatter(erows_a, didx_a, 2, ssem_a)

        wait_scatter(erows_a, ssem_a)
        wait_grp(sidx_b, didx_b, g0 + 1, isem_b)
        start_gather(sidx_b, 0, erows_a, gsem_a)
        wait_gather(erows_b, gsem_b)
        start_scatter(erows_b, didx_a, 3, ssem_b)

        wait_scatter(erows_b, ssem_b)

        @pl.when(k < NBODY - 1)
        def _():
            load_grp(sidx_a, didx_a, g0 + 2, isem_a)
        start_gather(sidx_b, 1, erows_b, gsem_b)
        wait_gather(erows_a, gsem_a)
        start_scatter(erows_a, didx_b, 0, ssem_a)

        wait_scatter(erows_a, ssem_a)
        start_gather(sidx_b, 2, erows_a, gsem_a)
        wait_gather(erows_b, gsem_b)
        start_scatter(erows_b, didx_b, 1, ssem_b)

        wait_scatter(erows_b, ssem_b)
        start_gather(sidx_b, 3, erows_b, gsem_b)
        wait_gather(erows_a, gsem_a)
        start_scatter(erows_a, didx_b, 2, ssem_a)
        return carry

    lax.fori_loop(0, NBODY, grp_body, 0)
    wait_gather(erows_b, gsem_b)
    start_scatter(erows_b, didx_b, 3, ssem_b)
    wait_scatter(erows_a, ssem_a)
    wait_scatter(erows_b, ssem_b)
    plsc.subcore_barrier()

    pltpu.sync_copy(comb_sh.at[pl.ds(s * NPT, NPT)],
                    comb_out.at[c, pl.ds(s * NPT, NPT)])


_sc_gnn = functools.partial(
    pl.kernel,
    out_type=(
        jax.ShapeDtypeStruct((NC, NPAD, EMB_D), jnp.float32),
        jax.ShapeDtypeStruct((NPAD, EMB_D), jnp.float32),
    ),
    mesh=plsc.VectorSubcoreMesh(
        core_axis_name="c", subcore_axis_name="s",
        num_cores=NC, num_subcores=NS),
    scratch_types=[
        pltpu.VMEM((NCHUNK,), jnp.int32),
        pltpu.VMEM((ECHUNK, EMB_D), jnp.float32),
        pltpu.VMEM((ECHUNK, EMB_D), jnp.float32),
        pltpu.VMEM((4, ECHUNK), jnp.int32),
        pltpu.VMEM((4, ECHUNK), jnp.int32),
        pltpu.VMEM((4, ECHUNK), jnp.int32),
        pltpu.VMEM((4, ECHUNK), jnp.int32),
        pltpu.VMEM_SHARED((NPAD, EMB_D), jnp.float32),
        pltpu.SemaphoreType.DMA,
        pltpu.SemaphoreType.DMA,
        pltpu.SemaphoreType.DMA,
        pltpu.SemaphoreType.DMA,
        pltpu.SemaphoreType.DMA,
        pltpu.SemaphoreType.DMA,
        pltpu.SemaphoreType.DMA,
        pltpu.SemaphoreType.DMA,
    ],
)(_sc_body)


BN = 2048
NBLK = NPAD // BN


def _tc_body(comb_ref, wmp_ref, bmp_ref, wcls_ref, bcls_ref, out_ref, acc_ref):
    i = pl.program_id(0)

    @pl.when(i == 0)
    def _():
        acc_ref[...] = jnp.zeros_like(acc_ref)

    cb = comb_ref[...]
    zin = cb[0] + cb[1]
    z = jax.lax.dot(zin, wmp_ref[...],
                    precision=jax.lax.Precision.HIGHEST,
                    preferred_element_type=jnp.float32)
    z = jnp.maximum(z + bmp_ref[...], 0.0)
    rid = i * BN + lax.broadcasted_iota(jnp.int32, (BN, 1), 0)
    z = jnp.where(rid < N_NODES, z, 0.0)
    acc_ref[...] += jnp.sum(z, axis=0, keepdims=True)

    @pl.when(i == NBLK - 1)
    def _():
        hg = acc_ref[...] * (1.0 / N_NODES)
        out_ref[...] = jax.lax.dot(
            hg, wcls_ref[...],
            precision=jax.lax.Precision.HIGHEST,
            preferred_element_type=jnp.float32) + bcls_ref[...]


def _tc_tail(comb, W_mp, b_mp, W_cls, b_cls):
    return pl.pallas_call(
        _tc_body,
        grid=(NBLK,),
        in_specs=[
            pl.BlockSpec((NC, BN, EMB_D), lambda i: (0, i, 0)),
            pl.BlockSpec((128, 128), lambda i: (0, 0)),
            pl.BlockSpec((1, 128), lambda i: (0, 0)),
            pl.BlockSpec((128, 16), lambda i: (0, 0)),
            pl.BlockSpec((1, 16), lambda i: (0, 0)),
        ],
        out_specs=pl.BlockSpec((1, 16), lambda i: (0, 0)),
        out_shape=jax.ShapeDtypeStruct((1, 16), jnp.float32),
        scratch_shapes=[pltpu.VMEM((1, 128), jnp.float32)],
    )(comb, W_mp, b_mp, W_cls, b_cls)


def kernel(x, edge_index, emb, W_mp, b_mp, W_cls, b_cls):
    x = x.astype(jnp.int32)
    pad = jnp.arange(NPAD - N_NODES, dtype=jnp.int32)
    xp = jnp.concatenate([x, pad])
    npe = EPAD - N_EDGES
    pe = jnp.arange(npe, dtype=jnp.int32)
    pads = jnp.stack([pe % N_NODES, N_NODES + pe % (NPAD - N_NODES)])
    es = jnp.concatenate([edge_index, pads], axis=1)
    es = es.reshape(2, EPAD // ECHUNK, ECHUNK)
    zer = jnp.zeros((NCHUNK, EMB_D), dtype=jnp.float32)
    comb, _h = _sc_gnn(xp, es, emb, zer)
    return _tc_tail(comb, W_mp, b_mp.reshape(1, 128), W_cls,
                    b_cls.reshape(1, 16))
